# Initial kernel scaffold; baseline (speedup 1.0000x reference)
import jax
import jax.numpy as jnp
from jax.experimental import pallas as pl


def kernel(z_nodes, z_edges, length, x_indices, mask_valid, params):
    raise NotImplementedError("write your pallas kernel here")


# speedup vs baseline: 3.4610x; 3.4610x over previous
import jax
import jax.numpy as jnp
from jax.experimental import pallas as pl

B, N, H, L = 64, 64, 64, 4
E = N * (N - 1) // 2
C_IN_N, C_IN_E, C_OUT_N, C_OUT_E = 16, 8, 16, 8
F32 = jnp.float32


def _dot(a, b):
    return jnp.dot(a, b, preferred_element_type=F32)


def _mlp2(x, w1, b1, w2, b2):
    return _dot(jax.nn.gelu(_dot(x, w1) + b1), w2) + b2


def _out_net(x, ln_g, ln_b, w1, b1, w2, b2):
    mu = jnp.mean(x, axis=-1, keepdims=True)
    var = jnp.mean((x - mu) ** 2, axis=-1, keepdims=True)
    h = (x - mu) / jnp.sqrt(var + 1e-5) * ln_g + ln_b
    return _dot(jax.nn.gelu(_dot(h, w1) + b1), w2) + b2


def _body(n_weights, *refs):
    (z_n_ref, z_e_ref, x1_ref, x2_ref, mrow_ref, mcol_ref) = refs[:6]
    w_refs = refs[6:6 + n_weights]
    out_n_ref, out_e_ref = refs[6 + n_weights:]

    ws = [r[...] for r in w_refs]
    cur = [0]

    def take(k):
        out = ws[cur[0]:cur[0] + k]
        cur[0] += k
        return out

    x1row = x1_ref[...]
    x2row = x2_ref[...]
    mrow = mrow_ref[0]
    mcol = mcol_ref[0]

    r_iota = jax.lax.broadcasted_iota(jnp.int32, (N, E), 0)
    g1t = (x1row == r_iota).astype(F32)
    g2t = (x2row == r_iota).astype(F32)
    a_m = (g1t + g2t) * mrow
    denom = jnp.clip(jnp.sum(a_m, axis=1, keepdims=True), 1.0, None)
    inv_denom = 1.0 / denom

    nodes = _mlp2(z_n_ref[0], *take(4))
    edges = _mlp2(z_e_ref[0], *take(4))

    for _ in range(L):
        agg = _dot(a_m, edges) * inv_denom
        u1a, u1b, u_b1, u2, u_b2 = take(5)
        hn = jax.nn.gelu(_dot(nodes, u1a) + _dot(agg, u1b) + u_b1)
        nodes = nodes + _dot(hn, u2) + u_b2
        v1a, v1b, v1c, v_b1, v2, v_b2 = take(6)
        t1 = _dot(nodes, v1a)
        t2 = _dot(nodes, v1b)
        g1n = jax.lax.dot_general(g1t, t1, (((0,), (0,)), ((), ())),
                                  preferred_element_type=F32)
        g2n = jax.lax.dot_general(g2t, t2, (((0,), (0,)), ((), ())),
                                  preferred_element_type=F32)
        he = jax.nn.gelu(g1n + g2n + _dot(edges, v1c) + v_b1)
        edges = edges + _dot(he, v2) + v_b2

    out_n_ref[0] = _out_net(nodes, *take(6))
    out_e_ref[0] = _out_net(edges, *take(6)) * mcol


def kernel(z_nodes, z_edges, length, x_indices, mask_valid, params):
    del length

    def row(v):
        return v.reshape(1, -1)

    ws = []
    for p in (params["in_n"], params["in_e"]):
        ws += [p["w1"], row(p["b1"]), p["w2"], row(p["b2"])]
    for lp in params["layers"]:
        u = lp["e2n"]
        ws += [u["w1"][:H], u["w1"][H:], row(u["b1"]), u["w2"], row(u["b2"])]
        v = lp["n2e"]
        ws += [v["w1"][:H], v["w1"][H:2 * H], v["w1"][2 * H:], row(v["b1"]),
               v["w2"], row(v["b2"])]
    for p in (params["out_n"], params["out_e"]):
        ws += [row(p["ln_g"]), row(p["ln_b"]), p["w1"], row(p["b1"]),
               p["w2"], row(p["b2"])]

    x1r = x_indices[0:1]
    x2r = x_indices[1:2]
    mrow = mask_valid.reshape(B, 1, E)
    mcol = mask_valid.reshape(B, E, 1)

    def const_spec(arr):
        nd = arr.ndim
        return pl.BlockSpec(arr.shape, lambda b, _n=nd: (0,) * _n)

    in_specs = [
        pl.BlockSpec((1, N, C_IN_N), lambda b: (b, 0, 0)),
        pl.BlockSpec((1, E, C_IN_E), lambda b: (b, 0, 0)),
        const_spec(x1r),
        const_spec(x2r),
        pl.BlockSpec((1, 1, E), lambda b: (b, 0, 0)),
        pl.BlockSpec((1, E, 1), lambda b: (b, 0, 0)),
    ] + [const_spec(w) for w in ws]

    out_specs = (
        pl.BlockSpec((1, N, C_OUT_N), lambda b: (b, 0, 0)),
        pl.BlockSpec((1, E, C_OUT_E), lambda b: (b, 0, 0)),
    )
    out_shape = (
        jax.ShapeDtypeStruct((B, N, C_OUT_N), F32),
        jax.ShapeDtypeStruct((B, E, C_OUT_E), F32),
    )

    import functools
    body = functools.partial(_body, len(ws))
    nodes_out, edges_out = pl.pallas_call(
        body,
        grid=(B,),
        in_specs=in_specs,
        out_specs=out_specs,
        out_shape=out_shape,
    )(z_nodes, z_edges, x1r, x2r, mrow, mcol, *ws)
    return nodes_out, edges_out


# speedup vs baseline: 4.7027x; 1.3588x over previous
import functools

import jax
import jax.numpy as jnp
from jax.experimental import pallas as pl

B, N, H, L = 64, 64, 64, 4
E = N * (N - 1) // 2
C_IN_N, C_IN_E, C_OUT_N, C_OUT_E = 16, 8, 16, 8
F32 = jnp.float32
BS = 4


def _dot(a, b):
    return jnp.dot(a, b, preferred_element_type=F32)


def _dot_t(a, b):
    return jax.lax.dot_general(a, b, (((0,), (0,)), ((), ())),
                               preferred_element_type=F32)


def _mlp2(x, w1, b1, w2, b2):
    return _dot(jax.nn.gelu(_dot(x, w1) + b1), w2) + b2


def _out_net(x, ln_g, ln_b, w1, b1, w2, b2):
    mu = jnp.mean(x, axis=-1, keepdims=True)
    var = jnp.mean((x - mu) ** 2, axis=-1, keepdims=True)
    h = (x - mu) / jnp.sqrt(var + 1e-5) * ln_g + ln_b
    return _dot(jax.nn.gelu(_dot(h, w1) + b1), w2) + b2


def _body(n_weights, *refs):
    (z_n_ref, z_e_ref, x1_ref, x2_ref, mrow_ref, mcol_ref) = refs[:6]
    w_refs = refs[6:6 + n_weights]
    out_n_ref, out_e_ref = refs[6 + n_weights:]

    ws = [r[...] for r in w_refs]
    cur = [0]

    def take(k):
        out = ws[cur[0]:cur[0] + k]
        cur[0] += k
        return out

    x1row = x1_ref[...]
    x2row = x2_ref[...]
    mrow = mrow_ref[0]
    mcol = mcol_ref[0]

    r_iota = jax.lax.broadcasted_iota(jnp.int32, (N, E), 0)
    g1t = (x1row == r_iota).astype(F32)
    g2t = (x2row == r_iota).astype(F32)
    g12t = jnp.concatenate([g1t, g2t], axis=0)
    a_u = g1t + g2t
    a_m = []
    inv_denom = []
    for g in range(BS):
        am = a_u * mrow[g:g + 1]
        a_m.append(am)
        inv_denom.append(1.0 / jnp.clip(jnp.sum(am, axis=1, keepdims=True),
                                        1.0, None))

    nodes = _mlp2(z_n_ref[0], *take(4))
    edges = _mlp2(z_e_ref[0], *take(4))

    for _ in range(L):
        agg = jnp.concatenate(
            [_dot(a_m[g], edges[g * E:(g + 1) * E]) * inv_denom[g]
             for g in range(BS)], axis=0)
        u1a, u1b, u_b1, u2, u_b2 = take(5)
        hn = jax.nn.gelu(_dot(nodes, u1a) + _dot(agg, u1b) + u_b1)
        nodes = nodes + _dot(hn, u2) + u_b2
        v1a, v1b, v1c, v_b1, v2, v_b2 = take(6)
        t1 = _dot(nodes, v1a)
        t2 = _dot(nodes, v1b)
        gn = jnp.concatenate(
            [_dot_t(g12t, jnp.concatenate(
                [t1[g * N:(g + 1) * N], t2[g * N:(g + 1) * N]], axis=0))
             for g in range(BS)], axis=0)
        he = jax.nn.gelu(gn + _dot(edges, v1c) + v_b1)
        edges = edges + _dot(he, v2) + v_b2

    out_n_ref[0] = _out_net(nodes, *take(6))
    out_e_ref[0] = _out_net(edges, *take(6)) * mcol


def kernel(z_nodes, z_edges, length, x_indices, mask_valid, params):
    del length

    def row(v):
        return v.reshape(1, -1)

    ws = []
    for p in (params["in_n"], params["in_e"]):
        ws += [p["w1"], row(p["b1"]), p["w2"], row(p["b2"])]
    for lp in params["layers"]:
        u = lp["e2n"]
        ws += [u["w1"][:H], u["w1"][H:], row(u["b1"]), u["w2"], row(u["b2"])]
        v = lp["n2e"]
        ws += [v["w1"][:H], v["w1"][H:2 * H], v["w1"][2 * H:], row(v["b1"]),
               v["w2"], row(v["b2"])]
    for p in (params["out_n"], params["out_e"]):
        ws += [row(p["ln_g"]), row(p["ln_b"]), p["w1"], row(p["b1"]),
               p["w2"], row(p["b2"])]

    nb = B // BS
    x1r = x_indices[0:1]
    x2r = x_indices[1:2]
    z_n = z_nodes.reshape(nb, BS * N, C_IN_N)
    z_e = z_edges.reshape(nb, BS * E, C_IN_E)
    mrow = mask_valid.reshape(nb, BS, E)
    mcol = mask_valid.reshape(nb, BS * E, 1)

    def const_spec(arr):
        nd = arr.ndim
        return pl.BlockSpec(arr.shape, lambda b, _n=nd: (0,) * _n)

    in_specs = [
        pl.BlockSpec((1, BS * N, C_IN_N), lambda b: (b, 0, 0)),
        pl.BlockSpec((1, BS * E, C_IN_E), lambda b: (b, 0, 0)),
        const_spec(x1r),
        const_spec(x2r),
        pl.BlockSpec((1, BS, E), lambda b: (b, 0, 0)),
        pl.BlockSpec((1, BS * E, 1), lambda b: (b, 0, 0)),
    ] + [const_spec(w) for w in ws]

    out_specs = (
        pl.BlockSpec((1, BS * N, C_OUT_N), lambda b: (b, 0, 0)),
        pl.BlockSpec((1, BS * E, C_OUT_E), lambda b: (b, 0, 0)),
    )
    out_shape = (
        jax.ShapeDtypeStruct((nb, BS * N, C_OUT_N), F32),
        jax.ShapeDtypeStruct((nb, BS * E, C_OUT_E), F32),
    )

    body = functools.partial(_body, len(ws))
    nodes_out, edges_out = pl.pallas_call(
        body,
        grid=(nb,),
        in_specs=in_specs,
        out_specs=out_specs,
        out_shape=out_shape,
    )(z_n, z_e, x1r, x2r, mrow, mcol, *ws)
    return (nodes_out.reshape(B, N, C_OUT_N),
            edges_out.reshape(B, E, C_OUT_E))


# speedup vs baseline: 5.9130x; 1.2574x over previous
import functools

import jax
import jax.numpy as jnp
from jax.experimental import pallas as pl

B, N, H, L = 64, 64, 64, 4
E = N * (N - 1) // 2
C_IN_N, C_IN_E, C_OUT_N, C_OUT_E = 16, 8, 16, 8
F32 = jnp.float32
BS = 4


def _dot(a, b):
    return jnp.dot(a, b, preferred_element_type=F32)


def _dot_t(a, b):
    return jax.lax.dot_general(a, b, (((0,), (0,)), ((), ())),
                               preferred_element_type=F32)


def _mlp2(x, w1, b1, w2, b2):
    return _dot(jax.nn.gelu(_dot(x, w1) + b1), w2) + b2


def _out_net(x, ln_g, ln_b, w1, b1, w2, b2):
    mu = jnp.mean(x, axis=-1, keepdims=True)
    var = jnp.mean((x - mu) ** 2, axis=-1, keepdims=True)
    h = (x - mu) / jnp.sqrt(var + 1e-5) * ln_g + ln_b
    return _dot(jax.nn.gelu(_dot(h, w1) + b1), w2) + b2


def _body(n_weights, *refs):
    (z_n_ref, z_e_ref, xi_ref, m_ref) = refs[:4]
    w_refs = refs[4:4 + n_weights]
    out_n_ref, out_e_ref = refs[4 + n_weights:]

    ws = [r[...] for r in w_refs]
    cur = [0]

    def take(k):
        out = ws[cur[0]:cur[0] + k]
        cur[0] += k
        return out

    def mlp2_w():
        w1, b1, w2, b2 = take(4)
        return w1, b1.reshape(1, -1), w2, b2.reshape(1, -1)

    xi = xi_ref[...]
    x1row = xi[0:1]
    x2row = xi[1:2]
    mrow = m_ref[0]

    r_iota = jax.lax.broadcasted_iota(jnp.int32, (N, E), 0)
    g1t = (x1row == r_iota).astype(F32)
    g2t = (x2row == r_iota).astype(F32)
    g12t = jnp.concatenate([g1t, g2t], axis=0)
    a_u = g1t + g2t
    a_m = []
    inv_denom = []
    for g in range(BS):
        am = a_u * mrow[g:g + 1]
        a_m.append(am)
        inv_denom.append(1.0 / jnp.clip(jnp.sum(am, axis=1, keepdims=True),
                                        1.0, None))

    nodes = _mlp2(z_n_ref[...].reshape(BS * N, C_IN_N), *mlp2_w())
    edges = _mlp2(z_e_ref[...].reshape(BS * E, C_IN_E), *mlp2_w())

    for _ in range(L):
        agg = jnp.concatenate(
            [_dot(a_m[g], edges[g * E:(g + 1) * E]) * inv_denom[g]
             for g in range(BS)], axis=0)
        u1, u_b1, u2, u_b2 = mlp2_w()
        hn = jax.nn.gelu(_dot(nodes, u1[:H]) + _dot(agg, u1[H:]) + u_b1)
        nodes = nodes + _dot(hn, u2) + u_b2
        v1, v_b1, v2, v_b2 = mlp2_w()
        t1 = _dot(nodes, v1[:H])
        t2 = _dot(nodes, v1[H:2 * H])
        gn = jnp.concatenate(
            [_dot_t(g12t, jnp.concatenate(
                [t1[g * N:(g + 1) * N], t2[g * N:(g + 1) * N]], axis=0))
             for g in range(BS)], axis=0)
        he = jax.nn.gelu(gn + _dot(edges, v1[2 * H:]) + v_b1)
        edges = edges + _dot(he, v2) + v_b2

    def out_w():
        ln_g, ln_b, w1, b1, w2, b2 = take(6)
        return (ln_g.reshape(1, -1), ln_b.reshape(1, -1), w1,
                b1.reshape(1, -1), w2, b2.reshape(1, -1))

    out_n_ref[...] = _out_net(nodes, *out_w()).reshape(BS, N, C_OUT_N)
    m_t = jnp.transpose(mrow)
    e_out = _out_net(edges, *out_w())
    for g in range(BS):
        out_e_ref[g] = e_out[g * E:(g + 1) * E] * m_t[:, g:g + 1]


def kernel(z_nodes, z_edges, length, x_indices, mask_valid, params):
    del length

    ws = []
    for p in (params["in_n"], params["in_e"]):
        ws += [p["w1"], p["b1"], p["w2"], p["b2"]]
    for lp in params["layers"]:
        for p in (lp["e2n"], lp["n2e"]):
            ws += [p["w1"], p["b1"], p["w2"], p["b2"]]
    for p in (params["out_n"], params["out_e"]):
        ws += [p["ln_g"], p["ln_b"], p["w1"], p["b1"], p["w2"], p["b2"]]

    nb = B // BS
    mask3 = mask_valid.reshape(nb, BS, E)

    def const_spec(arr):
        nd = arr.ndim
        return pl.BlockSpec(arr.shape, lambda b, _n=nd: (0,) * _n)

    in_specs = [
        pl.BlockSpec((BS, N, C_IN_N), lambda b: (b, 0, 0)),
        pl.BlockSpec((BS, E, C_IN_E), lambda b: (b, 0, 0)),
        const_spec(x_indices),
        pl.BlockSpec((1, BS, E), lambda b: (b, 0, 0)),
    ] + [const_spec(w) for w in ws]

    out_specs = (
        pl.BlockSpec((BS, N, C_OUT_N), lambda b: (b, 0, 0)),
        pl.BlockSpec((BS, E, C_OUT_E), lambda b: (b, 0, 0)),
    )
    out_shape = (
        jax.ShapeDtypeStruct((B, N, C_OUT_N), F32),
        jax.ShapeDtypeStruct((B, E, C_OUT_E), F32),
    )

    body = functools.partial(_body, len(ws))
    return pl.pallas_call(
        body,
        grid=(nb,),
        in_specs=in_specs,
        out_specs=out_specs,
        out_shape=out_shape,
    )(z_nodes, z_edges, x_indices, mask3, *ws)
